# baseline (speedup 1.0000x reference)
import functools

import jax
import jax.numpy as jnp
from jax import lax
from jax.experimental import pallas as pl
from jax.experimental.pallas import tpu as pltpu
from jax.experimental.pallas import tpu_sc as plsc

_B = 1024
_D = 3072
_N = 100000
_K = 10
_TN = 512
_U = 3
_FBIG = 33554432.0


def _insert(tv_ref, ti_ref, lane, m, g):
    del lane
    tv = tv_ref[...]
    ti = ti_ref[...]
    beats = (tv > m) | ((tv == m) & (ti < g))
    bf = beats.astype(jnp.float32)
    bs1 = jnp.concatenate(
        [jnp.ones((_B, 1), jnp.float32), bf[:, :-1]], axis=1) > 0.0
    stv = jnp.concatenate([tv[:, :1], tv[:, :-1]], axis=1)
    sti = jnp.concatenate([ti[:, :1], ti[:, :-1]], axis=1)
    tv_ref[...] = jnp.where(beats, tv, jnp.where(bs1, m, stv))
    ti_ref[...] = jnp.where(beats, ti, jnp.where(bs1, g, sti))


def _topk_body(q_ref, ka_ref, kb_ref, idx_out_ref, tv_ref, ti_ref,
               sca_ref, scb_ref, msa_ref, msb_ref):
    j = pl.program_id(0)
    nsteps = pl.num_programs(0)
    jf = j.astype(jnp.float32)

    @pl.when(j == 0)
    def _():
        tv_ref[...] = jnp.full((_B, 16), -jnp.inf, jnp.float32)
        ti_ref[...] = jnp.full((_B, 16), _FBIG, jnp.float32)

    lane = lax.broadcasted_iota(jnp.int32, (_B, 16), 1).astype(jnp.float32)
    iota = lax.broadcasted_iota(jnp.int32, (_B, _TN), 1).astype(jnp.float32)
    nt = pl.cdiv(_N, _TN)

    def merge(sc_ref, ms_ref, tile_f, valid):
        scores = sc_ref[...]
        gidx = iota + jnp.float32(_TN) * tile_f
        thresh = tv_ref[:, 9:10]
        qual = (scores > thresh) & (gidx < float(_N)) & valid
        ms_ref[...] = jnp.where(qual, scores, -jnp.inf)
        cnt = jnp.sum(qual.astype(jnp.float32), axis=1)
        rounds = jnp.minimum(jnp.max(cnt), float(_K)).astype(jnp.int32)
        m = g = None
        for r in range(_U):
            msv = ms_ref[...]
            if r == 0:
                e = msv
            else:
                e = jnp.where((msv < m) | ((msv == m) & (gidx > g)),
                              msv, -jnp.inf)
            m2 = jnp.max(e, axis=1, keepdims=True)
            g2 = jnp.min(jnp.where(e == m2, gidx, _FBIG), axis=1,
                         keepdims=True)
            _insert(tv_ref, ti_ref, lane, m2, g2)
            m, g = m2, g2
        return rounds, m, g, gidx

    def fallback(ms_ref, rounds, m, g, gidx):
        @pl.when(rounds > _U)
        def _():
            msv = ms_ref[...]
            ms_ref[...] = jnp.where(
                (msv < m) | ((msv == m) & (gidx > g)), msv, -jnp.inf)

            def body(_, carry):
                ms = ms_ref[...]
                mm = jnp.max(ms, axis=1, keepdims=True)
                gm = jnp.where(ms == mm, gidx, _FBIG)
                gg = jnp.min(gm, axis=1, keepdims=True)
                ms_ref[...] = jnp.where(gm == gg, -jnp.inf, ms)
                _insert(tv_ref, ti_ref, lane, mm, gg)
                return carry

            lax.fori_loop(0, rounds - _U, body, 0)

    ta = 2.0 * jf - 2.0
    tb = 2.0 * jf - 1.0
    valid_a = (j >= 1) & (2 * j - 2 < nt)
    valid_b = (j >= 1) & (2 * j - 1 < nt)
    ra, ma, ga, gia = merge(sca_ref, msa_ref, ta, valid_a)
    rb, mb, gb, gib = merge(scb_ref, msb_ref, tb, valid_b)

    ka = ka_ref[...]
    na = jnp.sqrt(jnp.sum(ka * ka, axis=1, keepdims=True))
    ka_ref[...] = ka / jnp.clip(na, 1e-12, None)
    sca_ref[...] = lax.dot_general(
        q_ref[...], ka_ref[...], (((1,), (1,)), ((), ())),
        preferred_element_type=jnp.float32)
    kb = kb_ref[...]
    nb = jnp.sqrt(jnp.sum(kb * kb, axis=1, keepdims=True))
    kb_ref[...] = kb / jnp.clip(nb, 1e-12, None)
    scb_ref[...] = lax.dot_general(
        q_ref[...], kb_ref[...], (((1,), (1,)), ((), ())),
        preferred_element_type=jnp.float32)

    fallback(msa_ref, ra, ma, ga, gia)
    fallback(msb_ref, rb, mb, gb, gib)

    @pl.when(j == nsteps - 1)
    def _():
        idx_out_ref[...] = ti_ref[...].astype(jnp.int32)


def _topk_indices(q_norm, keys):
    nt = pl.cdiv(_N, _TN)
    nsteps = (nt + 1) // 2 + 1
    idx16 = pl.pallas_call(
        _topk_body,
        grid=(nsteps,),
        in_specs=[
            pl.BlockSpec((_B, _D), lambda j: (0, 0)),
            pl.BlockSpec((_TN, _D), lambda j: (jnp.minimum(2 * j, nt - 1), 0)),
            pl.BlockSpec((_TN, _D),
                         lambda j: (jnp.minimum(2 * j + 1, nt - 1), 0)),
        ],
        out_specs=pl.BlockSpec((_B, 16), lambda j: (0, 0)),
        out_shape=jax.ShapeDtypeStruct((_B, 16), jnp.int32),
        scratch_shapes=[
            pltpu.VMEM((_B, 16), jnp.float32),
            pltpu.VMEM((_B, 16), jnp.float32),
            pltpu.VMEM((_B, _TN), jnp.float32),
            pltpu.VMEM((_B, _TN), jnp.float32),
            pltpu.VMEM((_B, _TN), jnp.float32),
            pltpu.VMEM((_B, _TN), jnp.float32),
        ],
    )(q_norm, keys, keys)
    return idx16[:, :_K]


def _sc_gather(keys, idx_flat):
    nw = 32
    bpw = idx_flat.shape[0] // nw
    ch = 16
    nch = bpw // ch
    mesh = plsc.VectorSubcoreMesh(core_axis_name="c", subcore_axis_name="s")

    @functools.partial(
        pl.kernel, mesh=mesh,
        out_type=jax.ShapeDtypeStruct((idx_flat.shape[0], _D), jnp.float32),
        scratch_types=[
            pltpu.VMEM((bpw,), jnp.int32),
            pltpu.VMEM((ch, _D), jnp.float32),
            pltpu.VMEM((ch, _D), jnp.float32),
            pltpu.SemaphoreType.DMA,
            pltpu.SemaphoreType.DMA,
        ],
    )
    def gk(table_hbm, idx_hbm, out_hbm, idx_v, rows0, rows1, sem0, sem1):
        wid = lax.axis_index("s") * 2 + lax.axis_index("c")
        base = wid * bpw
        pltpu.sync_copy(idx_hbm.at[pl.ds(base, bpw)], idx_v)

        pltpu.async_copy(table_hbm.at[idx_v.at[pl.ds(0, ch)]], rows0, sem0)

        def body(p, carry):
            c = 2 * p
            pltpu.async_copy(
                table_hbm.at[idx_v.at[pl.ds((c + 1) * ch, ch)]], rows1, sem1)
            pltpu.make_async_copy(
                table_hbm.at[idx_v.at[pl.ds(c * ch, ch)]], rows0, sem0).wait()
            pltpu.sync_copy(rows0, out_hbm.at[pl.ds(base + c * ch, ch)])

            @pl.when(c + 2 < nch)
            def _():
                pltpu.async_copy(
                    table_hbm.at[idx_v.at[pl.ds((c + 2) * ch, ch)]],
                    rows0, sem0)

            pltpu.make_async_copy(
                table_hbm.at[idx_v.at[pl.ds((c + 1) * ch, ch)]],
                rows1, sem1).wait()
            pltpu.sync_copy(rows1, out_hbm.at[pl.ds(base + (c + 1) * ch, ch)])
            return carry

        lax.fori_loop(0, nch // 2, body, 0)

    return gk(keys, idx_flat)


def kernel(query, keys, k):
    del k
    q = jnp.nan_to_num(query, nan=0.0, posinf=1.0, neginf=-1.0)
    qn = jnp.linalg.norm(q, ord=2, axis=-1, keepdims=True)
    q_norm = q / jnp.clip(qn, 1e-12, None)

    topk_idx = _topk_indices(q_norm, keys)
    emb = _sc_gather(keys, topk_idx.reshape(-1)).reshape(_B, _K, _D)
    return emb, topk_idx


# speedup vs baseline: 1.0520x; 1.0520x over previous
import functools

import jax
import jax.numpy as jnp
from jax import lax
from jax.experimental import pallas as pl
from jax.experimental.pallas import tpu as pltpu
from jax.experimental.pallas import tpu_sc as plsc

_B = 1024
_D = 3072
_N = 100000
_K = 10
_TN = 512
_U = 3
_FBIG = 33554432.0


def _insert(tv_ref, ti_ref, lane, m, g):
    tv = tv_ref[...]
    ti = ti_ref[...]
    beats = (tv > m) | ((tv == m) & (ti < g))
    pos = jnp.sum((beats & (lane < float(_K))).astype(jnp.float32),
                  axis=1, keepdims=True)
    stv = jnp.concatenate([tv[:, :1], tv[:, :-1]], axis=1)
    sti = jnp.concatenate([ti[:, :1], ti[:, :-1]], axis=1)
    tv_ref[...] = jnp.where(lane < pos, tv, jnp.where(lane == pos, m, stv))
    ti_ref[...] = jnp.where(lane < pos, ti, jnp.where(lane == pos, g, sti))


def _topk_body(q_ref, ka_ref, kb_ref, idx_out_ref, tv_ref, ti_ref,
               sca_ref, scb_ref, msa_ref, msb_ref):
    j = pl.program_id(0)
    nsteps = pl.num_programs(0)
    jf = j.astype(jnp.float32)

    @pl.when(j == 0)
    def _():
        tv_ref[...] = jnp.full((_B, 16), -jnp.inf, jnp.float32)
        ti_ref[...] = jnp.full((_B, 16), _FBIG, jnp.float32)

    lane = lax.broadcasted_iota(jnp.int32, (_B, 16), 1).astype(jnp.float32)
    iota = lax.broadcasted_iota(jnp.int32, (_B, _TN), 1).astype(jnp.float32)
    nt = pl.cdiv(_N, _TN)

    def merge(sc_ref, ms_ref, tile_f, valid):
        scores = sc_ref[...]
        gidx = iota + jnp.float32(_TN) * tile_f
        thresh = tv_ref[:, 9:10]
        qual = (scores > thresh) & (gidx < float(_N)) & valid
        ms_ref[...] = jnp.where(qual, scores, -jnp.inf)
        cnt = jnp.sum(qual.astype(jnp.float32), axis=1)
        rounds = jnp.minimum(jnp.max(cnt), float(_K)).astype(jnp.int32)
        m = g = None
        for r in range(_U):
            msv = ms_ref[...]
            if r == 0:
                e = msv
            else:
                e = jnp.where((msv < m) | ((msv == m) & (gidx > g)),
                              msv, -jnp.inf)
            m2 = jnp.max(e, axis=1, keepdims=True)
            g2 = jnp.min(jnp.where(e == m2, gidx, _FBIG), axis=1,
                         keepdims=True)
            _insert(tv_ref, ti_ref, lane, m2, g2)
            m, g = m2, g2
        return rounds, m, g, gidx

    def fallback(ms_ref, rounds, m, g, gidx):
        @pl.when(rounds > _U)
        def _():
            msv = ms_ref[...]
            ms_ref[...] = jnp.where(
                (msv < m) | ((msv == m) & (gidx > g)), msv, -jnp.inf)

            def body(_, carry):
                ms = ms_ref[...]
                mm = jnp.max(ms, axis=1, keepdims=True)
                gm = jnp.where(ms == mm, gidx, _FBIG)
                gg = jnp.min(gm, axis=1, keepdims=True)
                ms_ref[...] = jnp.where(gm == gg, -jnp.inf, ms)
                _insert(tv_ref, ti_ref, lane, mm, gg)
                return carry

            lax.fori_loop(0, rounds - _U, body, 0)

    ta = 2.0 * jf - 2.0
    tb = 2.0 * jf - 1.0
    valid_a = (j >= 1) & (2 * j - 2 < nt)
    valid_b = (j >= 1) & (2 * j - 1 < nt)
    ra, ma, ga, gia = merge(sca_ref, msa_ref, ta, valid_a)
    rb, mb, gb, gib = merge(scb_ref, msb_ref, tb, valid_b)

    ka = ka_ref[...]
    na = jnp.sqrt(jnp.sum(ka * ka, axis=1, keepdims=True))
    ka_ref[...] = ka / jnp.clip(na, 1e-12, None)
    sca_ref[...] = lax.dot_general(
        q_ref[...], ka_ref[...], (((1,), (1,)), ((), ())),
        preferred_element_type=jnp.float32)
    kb = kb_ref[...]
    nb = jnp.sqrt(jnp.sum(kb * kb, axis=1, keepdims=True))
    kb_ref[...] = kb / jnp.clip(nb, 1e-12, None)
    scb_ref[...] = lax.dot_general(
        q_ref[...], kb_ref[...], (((1,), (1,)), ((), ())),
        preferred_element_type=jnp.float32)

    fallback(msa_ref, ra, ma, ga, gia)
    fallback(msb_ref, rb, mb, gb, gib)

    @pl.when(j == nsteps - 1)
    def _():
        idx_out_ref[...] = ti_ref[...].astype(jnp.int32)


def _topk_indices(q_norm, keys):
    nt = pl.cdiv(_N, _TN)
    nsteps = (nt + 1) // 2 + 1
    idx16 = pl.pallas_call(
        _topk_body,
        grid=(nsteps,),
        in_specs=[
            pl.BlockSpec((_B, _D), lambda j: (0, 0)),
            pl.BlockSpec((_TN, _D), lambda j: (jnp.minimum(2 * j, nt - 1), 0)),
            pl.BlockSpec((_TN, _D),
                         lambda j: (jnp.minimum(2 * j + 1, nt - 1), 0)),
        ],
        out_specs=pl.BlockSpec((_B, 16), lambda j: (0, 0)),
        out_shape=jax.ShapeDtypeStruct((_B, 16), jnp.int32),
        scratch_shapes=[
            pltpu.VMEM((_B, 16), jnp.float32),
            pltpu.VMEM((_B, 16), jnp.float32),
            pltpu.VMEM((_B, _TN), jnp.float32),
            pltpu.VMEM((_B, _TN), jnp.float32),
            pltpu.VMEM((_B, _TN), jnp.float32),
            pltpu.VMEM((_B, _TN), jnp.float32),
        ],
    )(q_norm, keys, keys)
    return idx16[:, :_K]


def _sc_gather(keys, idx_flat):
    nw = 32
    bpw = idx_flat.shape[0] // nw
    ch = 16
    nch = bpw // ch
    mesh = plsc.VectorSubcoreMesh(core_axis_name="c", subcore_axis_name="s")

    @functools.partial(
        pl.kernel, mesh=mesh,
        out_type=jax.ShapeDtypeStruct((idx_flat.shape[0], _D), jnp.float32),
        scratch_types=[
            pltpu.VMEM((bpw,), jnp.int32),
            pltpu.VMEM((ch, _D), jnp.float32),
            pltpu.VMEM((ch, _D), jnp.float32),
            pltpu.SemaphoreType.DMA,
            pltpu.SemaphoreType.DMA,
        ],
    )
    def gk(table_hbm, idx_hbm, out_hbm, idx_v, rows0, rows1, sem0, sem1):
        wid = lax.axis_index("s") * 2 + lax.axis_index("c")
        base = wid * bpw
        pltpu.sync_copy(idx_hbm.at[pl.ds(base, bpw)], idx_v)

        pltpu.async_copy(table_hbm.at[idx_v.at[pl.ds(0, ch)]], rows0, sem0)

        def body(p, carry):
            c = 2 * p
            pltpu.async_copy(
                table_hbm.at[idx_v.at[pl.ds((c + 1) * ch, ch)]], rows1, sem1)
            pltpu.make_async_copy(
                table_hbm.at[idx_v.at[pl.ds(c * ch, ch)]], rows0, sem0).wait()
            pltpu.sync_copy(rows0, out_hbm.at[pl.ds(base + c * ch, ch)])

            @pl.when(c + 2 < nch)
            def _():
                pltpu.async_copy(
                    table_hbm.at[idx_v.at[pl.ds((c + 2) * ch, ch)]],
                    rows0, sem0)

            pltpu.make_async_copy(
                table_hbm.at[idx_v.at[pl.ds((c + 1) * ch, ch)]],
                rows1, sem1).wait()
            pltpu.sync_copy(rows1, out_hbm.at[pl.ds(base + (c + 1) * ch, ch)])
            return carry

        lax.fori_loop(0, nch // 2, body, 0)

    return gk(keys, idx_flat)


def kernel(query, keys, k):
    del k
    q = jnp.nan_to_num(query, nan=0.0, posinf=1.0, neginf=-1.0)
    qn = jnp.linalg.norm(q, ord=2, axis=-1, keepdims=True)
    q_norm = q / jnp.clip(qn, 1e-12, None)

    topk_idx = _topk_indices(q_norm, keys)
    emb = _sc_gather(keys, topk_idx.reshape(-1)).reshape(_B, _K, _D)
    return emb, topk_idx
